# Initial kernel scaffold; baseline (speedup 1.0000x reference)
import jax
import jax.numpy as jnp
from jax.experimental import pallas as pl


def kernel(encoded_sents, indices, hiddens, keys, U, V, W):
    raise NotImplementedError("write your pallas kernel here")


# speedup vs baseline: 1.7761x; 1.7761x over previous
import functools

import jax
import jax.numpy as jnp
from jax.experimental import pallas as pl
from jax.experimental.pallas import tpu as pltpu

_BATCH = 1024
_ENT = 64
_DIM = 256
_CURR = 512
_RB = 8


def _fused_body(starts_ref, order_ref,
                h_ref, k_ref, s_ref, u_ref, v_ref, w_ref,
                o_ref,
                sw_ref, uv_ref):
    i = pl.program_id(0)

    @pl.when(i == 0)
    def _():
        uv_ref[...] = u_ref[...] + v_ref[...]
        sw_ref[...] = jnp.dot(s_ref[...], w_ref[...],
                              preferred_element_type=jnp.float32)

    h = h_ref[...]
    hf = h.reshape(_RB * _ENT, _DIM)
    m = jnp.dot(hf, uv_ref[...], preferred_element_type=jnp.float32)

    for r in range(_RB):
        b = i * _RB + r
        lo = starts_ref[b]
        hi = starts_ref[b + 1]
        h_r = h[r]
        hk = h_r + k_ref[r]
        m_r = m[r * _ENT:(r + 1) * _ENT]

        def seg_body(j, acc, hk=hk, m_r=m_r):
            c = order_ref[j]
            s_c = s_ref[pl.ds(c, 1), :]
            gate = jax.nn.sigmoid(
                jnp.sum(hk * s_c, axis=1, keepdims=True))
            sw = sw_ref[pl.ds((c % 8) * _ENT, _ENT), :]
            return acc + gate * jnp.maximum(m_r + sw, 0.0)

        acc = jax.lax.fori_loop(lo, hi, seg_body, h_r)
        sq = jnp.sum(acc * acc, axis=1, keepdims=True)
        o_ref[r] = acc * jax.lax.rsqrt(jnp.maximum(sq, 1e-12))


@functools.partial(jax.jit, static_argnames=("interpret",))
def _run(encoded_sents, indices, hiddens, keys, U, V, W, interpret=False):
    indices = indices.astype(jnp.int32)
    counts = jnp.zeros((_BATCH,), jnp.int32).at[indices].add(1)
    starts = jnp.concatenate(
        [jnp.zeros((1,), jnp.int32), jnp.cumsum(counts, dtype=jnp.int32)])
    order = jnp.argsort(indices).astype(jnp.int32)

    grid_spec = pltpu.PrefetchScalarGridSpec(
        num_scalar_prefetch=2,
        grid=(_BATCH // _RB,),
        in_specs=[
            pl.BlockSpec((_RB, _ENT, _DIM), lambda i, *_: (i, 0, 0)),
            pl.BlockSpec((_RB, _ENT, _DIM), lambda i, *_: (i, 0, 0)),
            pl.BlockSpec((_CURR, _DIM), lambda i, *_: (0, 0)),
            pl.BlockSpec((_DIM, _DIM), lambda i, *_: (0, 0)),
            pl.BlockSpec((_DIM, _DIM), lambda i, *_: (0, 0)),
            pl.BlockSpec((_DIM, _DIM), lambda i, *_: (0, 0)),
        ],
        out_specs=pl.BlockSpec((_RB, _ENT, _DIM), lambda i, *_: (i, 0, 0)),
        scratch_shapes=[
            pltpu.VMEM((_CURR, _DIM), jnp.float32),
            pltpu.VMEM((_DIM, _DIM), jnp.float32),
        ],
    )
    return pl.pallas_call(
        _fused_body,
        grid_spec=grid_spec,
        out_shape=jax.ShapeDtypeStruct((_BATCH, _ENT, _DIM), jnp.float32),
        interpret=interpret,
    )(starts, order, hiddens, keys, encoded_sents, U, V, W)


def kernel(encoded_sents, indices, hiddens, keys, U, V, W):
    return _run(encoded_sents, indices, hiddens, keys, U, V, W)


# speedup vs baseline: 2.1499x; 1.2105x over previous
import functools

import jax
import jax.numpy as jnp
from jax.experimental import pallas as pl
from jax.experimental.pallas import tpu as pltpu

_BATCH = 1024
_ENT = 64
_DIM = 256
_CURR = 512
_RB = 8
_RF = _RB * _ENT


def _fused_body(starts_ref, order_ref, rows_ref,
                h_ref, k_ref, s_ref, u_ref, v_ref, w_ref,
                o_ref,
                sw_ref, uvb_ref, acc_ref, hk_ref, m_ref):
    i = pl.program_id(0)

    @pl.when(i == 0)
    def _():
        uvb_ref[...] = (u_ref[...] + v_ref[...]).astype(jnp.bfloat16)
        sw_ref[...] = jnp.dot(s_ref[...], w_ref[...],
                              preferred_element_type=jnp.float32)

    hf = h_ref[...].reshape(_RF, _DIM)
    acc_ref[...] = hf
    hk_ref[...] = hf + k_ref[...].reshape(_RF, _DIM)
    m_ref[...] = jnp.dot(hf.astype(jnp.bfloat16), uvb_ref[...],
                         preferred_element_type=jnp.float32)

    lo = starts_ref[i * _RB]
    hi = starts_ref[i * _RB + _RB]

    def seg_body(j, carry):
        c = order_ref[j]
        off = (rows_ref[j] - i * _RB) * _ENT
        s_c = s_ref[pl.ds(c, 1), :]
        hk = hk_ref[pl.ds(off, _ENT), :]
        gate = jax.nn.sigmoid(jax.lax.dot_general(
            hk, s_c, (((1,), (1,)), ((), ())),
            preferred_element_type=jnp.float32))
        sw = sw_ref[pl.ds((c % 8) * _ENT, _ENT), :]
        m = m_ref[pl.ds(off, _ENT), :]
        acc_ref[pl.ds(off, _ENT), :] += gate * jnp.maximum(m + sw, 0.0)
        return carry

    jax.lax.fori_loop(lo, hi, seg_body, 0, unroll=False)

    a = acc_ref[...]
    sq = jnp.sum(a * a, axis=1, keepdims=True)
    o_ref[...] = (a * jax.lax.rsqrt(jnp.maximum(sq, 1e-12))).reshape(
        _RB, _ENT, _DIM)


@functools.partial(jax.jit, static_argnames=("interpret",))
def _run(encoded_sents, indices, hiddens, keys, U, V, W, interpret=False):
    indices = indices.astype(jnp.int32)
    counts = jnp.zeros((_BATCH,), jnp.int32).at[indices].add(1)
    starts = jnp.concatenate(
        [jnp.zeros((1,), jnp.int32), jnp.cumsum(counts, dtype=jnp.int32)])
    order = jnp.argsort(indices).astype(jnp.int32)
    rows = indices[order]

    grid_spec = pltpu.PrefetchScalarGridSpec(
        num_scalar_prefetch=3,
        grid=(_BATCH // _RB,),
        in_specs=[
            pl.BlockSpec((_RB, _ENT, _DIM), lambda i, *_: (i, 0, 0)),
            pl.BlockSpec((_RB, _ENT, _DIM), lambda i, *_: (i, 0, 0)),
            pl.BlockSpec((_CURR, _DIM), lambda i, *_: (0, 0)),
            pl.BlockSpec((_DIM, _DIM), lambda i, *_: (0, 0)),
            pl.BlockSpec((_DIM, _DIM), lambda i, *_: (0, 0)),
            pl.BlockSpec((_DIM, _DIM), lambda i, *_: (0, 0)),
        ],
        out_specs=pl.BlockSpec((_RB, _ENT, _DIM), lambda i, *_: (i, 0, 0)),
        scratch_shapes=[
            pltpu.VMEM((_CURR, _DIM), jnp.float32),
            pltpu.VMEM((_DIM, _DIM), jnp.bfloat16),
            pltpu.VMEM((_RF, _DIM), jnp.float32),
            pltpu.VMEM((_RF, _DIM), jnp.float32),
            pltpu.VMEM((_RF, _DIM), jnp.float32),
        ],
    )
    return pl.pallas_call(
        _fused_body,
        grid_spec=grid_spec,
        out_shape=jax.ShapeDtypeStruct((_BATCH, _ENT, _DIM), jnp.float32),
        interpret=interpret,
    )(starts, order, rows, hiddens, keys, encoded_sents, U, V, W)


def kernel(encoded_sents, indices, hiddens, keys, U, V, W):
    return _run(encoded_sents, indices, hiddens, keys, U, V, W)
